# baseline (speedup 1.0000x reference)
import functools

import jax
import jax.numpy as jnp
from jax import lax
from jax.experimental import pallas as pl
from jax.experimental.pallas import tpu as pltpu
from jax.experimental.pallas import tpu_sc as plsc

KB = 1024
NC = 21
NBINS = NC * 2 * KB
NW = 32
CHUNK = 1024
NCH = 16
CW = NC * CHUNK


def _bin_kernel(x_ref, lab_ref, out_ref):
    x = x_ref[0]
    m = jnp.max(x, axis=0)
    e = jnp.exp(x - m[None])
    s = jnp.sum(e, axis=0)
    p = e * (1.0 / s)[None]
    lab = lab_ref[0]
    cls = lax.broadcasted_iota(jnp.int32, (NC,) + lab.shape, 0)
    fg = lab[None] == cls
    err = jnp.where(fg, 1.0 - p, p)
    b = jnp.minimum((err * KB).astype(jnp.int32), KB - 1)
    pk = cls * (2 * KB) + jnp.where(fg, KB, 0) + b
    packed = pk[:, :, :256] | (pk[:, :, 256:] << 16)
    out_ref[0, 0] = packed[:, :, :128]
    out_ref[0, 1] = packed[:, :, 128:]


def _make_sc_hist_body(nch):
    def _sc_hist_body(pk_hbm, out_hbm, buf_a, buf_b, hist, sem):
        wid = lax.axis_index("s") * 2 + lax.axis_index("c")
        zeros16 = jnp.zeros((16,), jnp.int32)

        def zbody(j, carry):
            hist[pl.ds(j * 16, 16)] = zeros16
            return carry

        lax.fori_loop(0, NBINS // 16, zbody, 0)

        ones16 = jnp.ones((16,), jnp.int32)
        lanes = lax.iota(jnp.int32, 16)
        diag0 = lanes * jnp.int32(CHUNK + 1)
        base = wid * nch * CW

        def issue(ci, buf):
            return pltpu.async_copy(
                pk_hbm.at[pl.ds(base + ci * CW, CW)], buf, sem)

        def process(buf):
            def gbody(g, c2):
                b = diag0 + g * 16
                idxs = []
                for c in range(NC):
                    if c == 0:
                        idxs.append(b)
                    else:
                        b = b + jnp.int32(CHUNK)
                        idxs.append(
                            jnp.where(lanes + jnp.int32(c) >= jnp.int32(NC),
                                      b - jnp.int32(CW), b))
                vs = [plsc.load_gather(buf, [idx]) for idx in idxs]
                for v in vs:
                    plsc.addupdate_scatter(
                        hist, [v & jnp.int32(0xFFFF)], ones16)
                    plsc.addupdate_scatter(
                        hist, [lax.shift_right_logical(v, 16)], ones16)
                return c2

            lax.fori_loop(0, CHUNK // 16, gbody, 0)

        issue(0, buf_a)

        def chunk_body(i, carry):
            for par, (cur, nxt) in enumerate([(buf_a, buf_b), (buf_b, buf_a)]):
                ci = i * 2 + par
                issue(jnp.minimum(ci + 1, nch - 1), nxt)
                pltpu.make_async_copy(
                    pk_hbm.at[pl.ds(base, CW)], cur, sem).wait()
                process(cur)
            return carry

        lax.fori_loop(0, nch // 2, chunk_body, 0)
        pltpu.make_async_copy(pk_hbm.at[pl.ds(base, CW)], buf_a, sem).wait()
        pltpu.sync_copy(hist, out_hbm.at[wid])

    return _sc_hist_body


def _final_kernel(h_ref, o_ref):
    h = jnp.sum(h_ref[...].astype(jnp.float32), axis=0)
    m_f = h[:, KB:]
    m = h[:, :KB] + m_f
    a = jnp.concatenate([m, m_f], axis=0)
    ii = lax.broadcasted_iota(jnp.int32, (KB, KB), 0)
    jj = lax.broadcasted_iota(jnp.int32, (KB, KB), 1)
    tri = (ii >= jj).astype(jnp.float32)
    s = lax.dot_general(a, tri, (((1,), (0,)), ((), ())),
                        preferred_element_type=jnp.float32)
    s_m, s_f = s[:NC], s[NC:]
    g = s_f[:, 0:1]
    j_in = 1.0 - (g - s_f) / jnp.maximum(g + s_m - s_f, 1.0)
    sme, sfe = s_m - m, s_f - m_f
    j_ex = 1.0 - (g - sfe) / jnp.maximum(g + sme - sfe, 1.0)
    mid = (lax.broadcasted_iota(jnp.int32, (1, KB), 1).astype(jnp.float32)
           + 0.5) / KB
    lc = jnp.sum(mid * (j_in - j_ex), axis=1, keepdims=True)
    present = (g > 0).astype(jnp.float32)
    num = jnp.sum(lc * present, axis=(0, 1), keepdims=True)
    den = jnp.sum(present, axis=(0, 1), keepdims=True)
    o_ref[...] = num / jnp.maximum(den, 1.0)


def kernel(inputs, targets):
    B, C, H, W = inputs.shape
    hb = H // 8
    nch_s = NCH // B
    mesh = plsc.VectorSubcoreMesh(core_axis_name="c", subcore_axis_name="s")
    sc_call = pl.kernel(
        _make_sc_hist_body(nch_s),
        out_type=jax.ShapeDtypeStruct((NW, NBINS), jnp.int32),
        mesh=mesh,
        compiler_params=pltpu.CompilerParams(
            needs_layout_passes=False, disable_bounds_checks=True),
        scratch_types=[
            pltpu.VMEM((CW,), jnp.int32),
            pltpu.VMEM((CW,), jnp.int32),
            pltpu.VMEM((NBINS,), jnp.int32),
            pltpu.SemaphoreType.DMA,
        ],
    )

    hists = []
    for s in range(B):
        pk = pl.pallas_call(
            _bin_kernel,
            grid=(hb,),
            in_specs=[
                pl.BlockSpec((1, C, 8, W), lambda h, s=s: (s, 0, h, 0)),
                pl.BlockSpec((1, 8, W), lambda h, s=s: (s, h, 0)),
            ],
            out_specs=pl.BlockSpec(
                (1, 2, C, 8, 128),
                lambda h: (h // 2, h % 2, 0, 0, 0)),
            out_shape=jax.ShapeDtypeStruct((NW, nch_s, C, 8, 128), jnp.int32),
        )(inputs, targets)
        hists.append(sc_call(pk.reshape(NW * nch_s * C * 8 * 128)))

    h_all = jnp.stack(hists).reshape(B * NW, NC, 2 * KB)
    loss = pl.pallas_call(
        _final_kernel,
        out_shape=jax.ShapeDtypeStruct((1, 1), jnp.float32),
    )(h_all)

    return loss[0, 0]


# speedup vs baseline: 1.8434x; 1.8434x over previous
import functools

import jax
import jax.numpy as jnp
from jax import lax
from jax.experimental import pallas as pl
from jax.experimental.pallas import tpu as pltpu
from jax.experimental.pallas import tpu_sc as plsc

KB = 1024
NC = 21
NBINS = NC * 2 * KB
NW = 32
CHUNK = 1024
NCH = 16
CW = NC * CHUNK


def _bin_kernel(x_ref, lab_ref, out_ref):
    hrows = x_ref.shape[2]
    x = x_ref[0]
    m = jnp.max(x, axis=0)
    e = jnp.exp(x - m[None])
    s = jnp.sum(e, axis=0)
    p = e * (1.0 / s)[None]
    lab = lab_ref[0]
    cls = lax.broadcasted_iota(jnp.int32, (NC,) + lab.shape, 0)
    fg = lab[None] == cls
    err = jnp.where(fg, 1.0 - p, p)
    b = jnp.minimum((err * KB).astype(jnp.int32), KB - 1)
    pk = cls * (2 * KB) + jnp.where(fg, KB, 0) + b
    for r in range(hrows // 8):
        blk = pk[:, r * 8:(r + 1) * 8]
        packed = blk[:, :, :256] | (blk[:, :, 256:] << 16)
        out_ref[0, 2 * r] = packed[:, :, :128]
        out_ref[0, 2 * r + 1] = packed[:, :, 128:]


def _make_sc_hist_body(nch):
    def _sc_hist_body(pk_hbm, out_hbm, buf_a, buf_b, hist, sem):
        wid = lax.axis_index("s") * 2 + lax.axis_index("c")
        zeros16 = jnp.zeros((16,), jnp.int32)

        def zbody(j, carry):
            hist[pl.ds(j * 16, 16)] = zeros16
            return carry

        lax.fori_loop(0, NBINS // 16, zbody, 0)

        ones16 = jnp.ones((16,), jnp.int32)
        lanes = lax.iota(jnp.int32, 16)
        diag0 = lanes * jnp.int32(CHUNK + 1)
        base = wid * nch * CW

        def issue(ci, buf):
            return pltpu.async_copy(
                pk_hbm.at[pl.ds(base + ci * CW, CW)], buf, sem)

        def process(buf):
            def gbody(g, c2):
                b = diag0 + g * 16
                idxs = []
                for c in range(NC):
                    if c == 0:
                        idxs.append(b)
                    else:
                        b = b + jnp.int32(CHUNK)
                        idxs.append(
                            jnp.where(lanes + jnp.int32(c) >= jnp.int32(NC),
                                      b - jnp.int32(CW), b))
                vs = [plsc.load_gather(buf, [idx]) for idx in idxs]
                for v in vs:
                    plsc.addupdate_scatter(
                        hist, [v & jnp.int32(0xFFFF)], ones16)
                    plsc.addupdate_scatter(
                        hist, [lax.shift_right_logical(v, 16)], ones16)
                return c2

            lax.fori_loop(0, CHUNK // 16, gbody, 0)

        issue(0, buf_a)

        def chunk_body(i, carry):
            for par, (cur, nxt) in enumerate([(buf_a, buf_b), (buf_b, buf_a)]):
                ci = i * 2 + par
                issue(jnp.minimum(ci + 1, nch - 1), nxt)
                pltpu.make_async_copy(
                    pk_hbm.at[pl.ds(base, CW)], cur, sem).wait()
                process(cur)
            return carry

        lax.fori_loop(0, nch // 2, chunk_body, 0)
        pltpu.make_async_copy(pk_hbm.at[pl.ds(base, CW)], buf_a, sem).wait()
        pltpu.sync_copy(hist, out_hbm.at[wid])

    return _sc_hist_body


def _final_kernel(h_ref, o_ref):
    h = jnp.sum(h_ref[...].astype(jnp.float32), axis=0)
    m_f = h[:, KB:]
    m = h[:, :KB] + m_f
    a = jnp.concatenate([m, m_f], axis=0)
    ii = lax.broadcasted_iota(jnp.int32, (KB, KB), 0)
    jj = lax.broadcasted_iota(jnp.int32, (KB, KB), 1)
    tri = (ii >= jj).astype(jnp.float32)
    s = lax.dot_general(a, tri, (((1,), (0,)), ((), ())),
                        preferred_element_type=jnp.float32)
    s_m, s_f = s[:NC], s[NC:]
    g = s_f[:, 0:1]
    j_in = 1.0 - (g - s_f) / jnp.maximum(g + s_m - s_f, 1.0)
    sme, sfe = s_m - m, s_f - m_f
    j_ex = 1.0 - (g - sfe) / jnp.maximum(g + sme - sfe, 1.0)
    mid = (lax.broadcasted_iota(jnp.int32, (1, KB), 1).astype(jnp.float32)
           + 0.5) / KB
    lc = jnp.sum(mid * (j_in - j_ex), axis=1, keepdims=True)
    present = (g > 0).astype(jnp.float32)
    num = jnp.sum(lc * present, axis=(0, 1), keepdims=True)
    den = jnp.sum(present, axis=(0, 1), keepdims=True)
    o_ref[...] = num / jnp.maximum(den, 1.0)


def kernel(inputs, targets):
    B, C, H, W = inputs.shape
    HB = 64
    hb = H // HB
    pk = pl.pallas_call(
        _bin_kernel,
        grid=(B, hb),
        in_specs=[
            pl.BlockSpec((1, C, HB, W), lambda b, h: (b, 0, h, 0)),
            pl.BlockSpec((1, HB, W), lambda b, h: (b, h, 0)),
        ],
        out_specs=pl.BlockSpec(
            (1, NCH, C, 8, 128),
            lambda b, h: (b * hb + h, 0, 0, 0, 0)),
        out_shape=jax.ShapeDtypeStruct((NW, NCH, C, 8, 128), jnp.int32),
    )(inputs, targets)

    pk = pk.reshape(NW * NCH * C * 8 * 128)

    mesh = plsc.VectorSubcoreMesh(core_axis_name="c", subcore_axis_name="s")
    hists = pl.kernel(
        _make_sc_hist_body(NCH),
        out_type=jax.ShapeDtypeStruct((NW, NBINS), jnp.int32),
        mesh=mesh,
        compiler_params=pltpu.CompilerParams(
            needs_layout_passes=False, disable_bounds_checks=True),
        scratch_types=[
            pltpu.VMEM((CW,), jnp.int32),
            pltpu.VMEM((CW,), jnp.int32),
            pltpu.VMEM((NBINS,), jnp.int32),
            pltpu.SemaphoreType.DMA,
        ],
    )(pk)

    loss = pl.pallas_call(
        _final_kernel,
        out_shape=jax.ShapeDtypeStruct((1, 1), jnp.float32),
    )(hists.reshape(NW, NC, 2 * KB))

    return loss[0, 0]
